# baseline (speedup 1.0000x reference)
import functools

import jax
import jax.numpy as jnp
from jax import lax
from jax.experimental import pallas as pl
from jax.experimental.pallas import tpu as pltpu
from jax.experimental.pallas import tpu_sc as plsc

B = 256
WENC = 4096
WDEC = 4096
MAXS = 2048
NC = 2
NS = 16
NW = NC * NS
ROWS = B // NW
BUF = WENC + 2 * MAXS
L = 16
CHUNKS = WENC // L


def _frames_sc(xe, xe_lens, xd, xt, xt_lens, prev):
    mesh = plsc.VectorSubcoreMesh(core_axis_name="c", subcore_axis_name="s")

    @functools.partial(
        pl.kernel,
        mesh=mesh,
        compiler_params=pltpu.CompilerParams(needs_layout_passes=False),
        out_type=[
            jax.ShapeDtypeStruct((B, WENC), jnp.int32),
            jax.ShapeDtypeStruct((B, WDEC), jnp.int32),
            jax.ShapeDtypeStruct((B, WENC), jnp.int32),
        ],
        scratch_types=[
            pltpu.VMEM((BUF,), jnp.int32),
            pltpu.VMEM((BUF,), jnp.int32),
            pltpu.VMEM((WENC,), jnp.int32),
            pltpu.VMEM((WENC,), jnp.int32),
            pltpu.VMEM((WENC,), jnp.int32),
            pltpu.VMEM((WENC,), jnp.int32),
            pltpu.VMEM((L,), jnp.int32),
            pltpu.VMEM((L,), jnp.int32),
            pltpu.VMEM((MAXS,), jnp.int32),
            pltpu.SemaphoreType.DMA,
            pltpu.SemaphoreType.DMA,
            pltpu.SemaphoreType.DMA,
            pltpu.SemaphoreType.DMA,
            pltpu.SemaphoreType.DMA,
        ],
    )
    def k(xe_h, el_h, xd_h, xt_h, tl_h, prev_h,
          ye_h, yd_h, p_h, buf0, buf1, ye0, ye1, p0, p1, el_v, tl_v, zero_v,
          sem_in0, sem_in1, sem_out0, sem_out1, sem_yd):
        wid = lax.axis_index("s") * NC + lax.axis_index("c")
        base = wid * ROWS
        lanes = lax.iota(jnp.int32, L)
        bufs = (buf0, buf1)
        yes = (ye0, ye1)
        ps = (p0, p1)
        sem_in = (sem_in0, sem_in1)
        sem_out = (sem_out0, sem_out1)

        zv = jnp.zeros((L,), jnp.int32)

        @pl.loop(0, MAXS // L)
        def _(i):
            zero_v[pl.ds(pl.multiple_of(i * L, L), L)] = zv

        pltpu.sync_copy(el_h.at[pl.ds(base, ROWS)], el_v.at[pl.ds(0, ROWS)])
        pltpu.sync_copy(tl_h.at[pl.ds(base, ROWS)], tl_v.at[pl.ds(0, ROWS)])
        el_vec = el_v[...]
        tl_vec = tl_v[...]

        yd_handles = []
        for r in range(ROWS):
            row = base + r
            yd_handles.append(
                pltpu.async_copy(xd_h.at[row], yd_h.at[row, pl.ds(0, MAXS)],
                                 sem_yd))
            yd_handles.append(
                pltpu.async_copy(zero_v, yd_h.at[row, pl.ds(MAXS, WDEC - MAXS)],
                                 sem_yd))

        def start_inputs(r):
            s = sem_in[r % 2]
            buf = bufs[r % 2]
            row = base + r
            return (
                pltpu.async_copy(prev_h.at[row], buf.at[pl.ds(0, WENC)], s),
                pltpu.async_copy(xe_h.at[row], buf.at[pl.ds(WENC, MAXS)], s),
                pltpu.async_copy(xt_h.at[row],
                                 buf.at[pl.ds(WENC + MAXS, MAXS)], s),
            )

        in_flight = {0: start_inputs(0)}
        out_flight = {}
        for r in range(ROWS):
            slot = r % 2
            row = base + r
            if r + 1 < ROWS:
                in_flight[r + 1] = start_inputs(r + 1)
            if r - 2 in out_flight:
                for h in out_flight.pop(r - 2):
                    h.wait()
            for h in in_flight.pop(r):
                h.wait()

            el = el_vec[r]
            tl = tl_vec[r]
            el_lanes = el + lanes
            tl_lanes = tl + lanes
            a_vec = tl_lanes + el
            b_vec = tl_lanes + MAXS
            buf = bufs[slot]
            yev = yes[slot]
            pv = ps[slot]

            @pl.loop(0, CHUNKS, unroll=8)
            def _(kk):
                off = pl.multiple_of(kk * L, L)
                yev[pl.ds(off, L)] = plsc.load_gather(buf, [el_lanes + off])
                q = tl_lanes + off
                idx2 = jnp.where(q < WENC, a_vec + off, b_vec + off)
                pv[pl.ds(off, L)] = plsc.load_gather(buf, [idx2])

            out_flight[r] = (
                pltpu.async_copy(yev, ye_h.at[row], sem_out[slot]),
                pltpu.async_copy(pv, p_h.at[row], sem_out[slot]),
            )

        for r, hs in sorted(out_flight.items()):
            for h in hs:
                h.wait()
        for h in yd_handles:
            h.wait()

    return k(xe, xe_lens, xd, xt, xt_lens, prev)


def kernel(xe, xe_lens, xd, xd_lens, xt, xt_lens, prev):
    el = xe_lens.astype(jnp.int32)
    dl = xd_lens.astype(jnp.int32)
    ye, yd, p = _frames_sc(xe, el, xd, xt, xt_lens.astype(jnp.int32), prev)
    return (ye, el, yd, dl, p)


# speedup vs baseline: 1.8413x; 1.8413x over previous
import functools

import jax
import jax.numpy as jnp
from jax import lax
from jax.experimental import pallas as pl
from jax.experimental.pallas import tpu as pltpu
from jax.experimental.pallas import tpu_sc as plsc

B = 256
WENC = 4096
WDEC = 4096
MAXS = 2048
NC = 2
NS = 16
NW = NC * NS
ROWS = B // NW
BUF = WENC + 2 * MAXS
L = 16
CHUNKS = WENC // L


def _frames_sc(xe, xe_lens, xd, xt, xt_lens, prev):
    mesh = plsc.VectorSubcoreMesh(core_axis_name="c", subcore_axis_name="s")

    @functools.partial(
        pl.kernel,
        mesh=mesh,
        compiler_params=pltpu.CompilerParams(needs_layout_passes=False),
        out_type=[
            jax.ShapeDtypeStruct((B, WENC), jnp.int32),
            jax.ShapeDtypeStruct((B, WDEC), jnp.int32),
            jax.ShapeDtypeStruct((B, WENC), jnp.int32),
        ],
        scratch_types=[
            pltpu.VMEM((BUF,), jnp.int32),
            pltpu.VMEM((BUF,), jnp.int32),
            pltpu.VMEM((WENC,), jnp.int32),
            pltpu.VMEM((WENC,), jnp.int32),
            pltpu.VMEM((WENC,), jnp.int32),
            pltpu.VMEM((WENC,), jnp.int32),
            pltpu.VMEM((L,), jnp.int32),
            pltpu.VMEM((L,), jnp.int32),
        ] + [pltpu.VMEM((WDEC,), jnp.int32) for _ in range(ROWS)]
          + [
            pltpu.SemaphoreType.DMA,
            pltpu.SemaphoreType.DMA,
            pltpu.SemaphoreType.DMA,
            pltpu.SemaphoreType.DMA,
            pltpu.SemaphoreType.DMA,
        ],
    )
    def k(xe_h, el_h, xd_h, xt_h, tl_h, prev_h,
          ye_h, yd_h, p_h, buf0, buf1, ye0, ye1, p0, p1, el_v, tl_v,
          yd0, yd1, yd2, yd3, yd4, yd5, yd6, yd7,
          sem_in0, sem_in1, sem_out0, sem_out1, sem_yd):
        wid = lax.axis_index("s") * NC + lax.axis_index("c")
        base = wid * ROWS
        lanes = lax.iota(jnp.int32, L)
        bufs = (buf0, buf1)
        yes = (ye0, ye1)
        ps = (p0, p1)
        yds = (yd0, yd1, yd2, yd3, yd4, yd5, yd6, yd7)
        sem_in = (sem_in0, sem_in1)
        sem_out = (sem_out0, sem_out1)

        xd_handles = [
            pltpu.async_copy(xd_h.at[base + r], yds[r].at[pl.ds(0, MAXS)],
                             sem_yd)
            for r in range(ROWS)
        ]
        zv = jnp.zeros((L,), jnp.int32)
        for r in range(ROWS):
            ydr = yds[r]

            @pl.loop(0, (WDEC - MAXS) // L)
            def _(i):
                ydr[pl.ds(pl.multiple_of(MAXS + i * L, L), L)] = zv

        pltpu.sync_copy(el_h.at[pl.ds(base, ROWS)], el_v.at[pl.ds(0, ROWS)])
        pltpu.sync_copy(tl_h.at[pl.ds(base, ROWS)], tl_v.at[pl.ds(0, ROWS)])
        el_vec = el_v[...]
        tl_vec = tl_v[...]

        def start_inputs(r):
            s = sem_in[r % 2]
            buf = bufs[r % 2]
            row = base + r
            return (
                pltpu.async_copy(prev_h.at[row], buf.at[pl.ds(0, WENC)], s),
                pltpu.async_copy(xe_h.at[row], buf.at[pl.ds(WENC, MAXS)], s),
                pltpu.async_copy(xt_h.at[row],
                                 buf.at[pl.ds(WENC + MAXS, MAXS)], s),
            )

        in_flight = {0: start_inputs(0)}
        out_flight = {}
        yd_out = []
        for r in range(ROWS):
            slot = r % 2
            row = base + r
            if r + 1 < ROWS:
                in_flight[r + 1] = start_inputs(r + 1)
            if r - 2 in out_flight:
                for h in out_flight.pop(r - 2):
                    h.wait()
            for h in in_flight.pop(r):
                h.wait()

            el = el_vec[r]
            tl = tl_vec[r]
            el_lanes = el + lanes
            tl_lanes = tl + lanes
            a_vec = tl_lanes + el
            b_vec = tl_lanes + MAXS
            buf = bufs[slot]
            yev = yes[slot]
            pv = ps[slot]

            @pl.loop(0, CHUNKS, unroll=8)
            def _(kk):
                off = pl.multiple_of(kk * L, L)
                yev[pl.ds(off, L)] = plsc.load_gather(buf, [el_lanes + off])
                q = tl_lanes + off
                idx2 = jnp.where(q < WENC, a_vec + off, b_vec + off)
                pv[pl.ds(off, L)] = plsc.load_gather(buf, [idx2])

            xd_handles[r].wait()
            yd_out.append(pltpu.async_copy(yds[r], yd_h.at[row], sem_yd))
            out_flight[r] = (
                pltpu.async_copy(yev, ye_h.at[row], sem_out[slot]),
                pltpu.async_copy(pv, p_h.at[row], sem_out[slot]),
            )

        for r, hs in sorted(out_flight.items()):
            for h in hs:
                h.wait()
        for h in yd_out:
            h.wait()

    return k(xe, xe_lens, xd, xt, xt_lens, prev)


def kernel(xe, xe_lens, xd, xd_lens, xt, xt_lens, prev):
    el = xe_lens.astype(jnp.int32)
    dl = xd_lens.astype(jnp.int32)
    ye, yd, p = _frames_sc(xe, el, xd, xt, xt_lens.astype(jnp.int32), prev)
    return (ye, el, yd, dl, p)


# speedup vs baseline: 1.9272x; 1.0467x over previous
import functools

import jax
import jax.numpy as jnp
from jax import lax
from jax.experimental import pallas as pl
from jax.experimental.pallas import tpu as pltpu
from jax.experimental.pallas import tpu_sc as plsc

B = 256
WENC = 4096
WDEC = 4096
MAXS = 2048
NC = 2
NS = 16
NW = NC * NS
ROWS = B // NW
BUF = WENC + 2 * MAXS
L = 16
CHUNKS = WENC // L
OSLOTS = 3


def _frames_sc(xe, xe_lens, xd, xt, xt_lens, prev):
    mesh = plsc.VectorSubcoreMesh(core_axis_name="c", subcore_axis_name="s")

    @functools.partial(
        pl.kernel,
        mesh=mesh,
        compiler_params=pltpu.CompilerParams(needs_layout_passes=False),
        out_type=[
            jax.ShapeDtypeStruct((B, WENC), jnp.int32),
            jax.ShapeDtypeStruct((B, WDEC), jnp.int32),
            jax.ShapeDtypeStruct((B, WENC), jnp.int32),
        ],
        scratch_types=(
            [pltpu.VMEM((BUF,), jnp.int32) for _ in range(ROWS)]
            + [pltpu.VMEM((WENC,), jnp.int32) for _ in range(OSLOTS)]
            + [pltpu.VMEM((WENC,), jnp.int32) for _ in range(OSLOTS)]
            + [pltpu.VMEM((WDEC,), jnp.int32) for _ in range(ROWS)]
            + [
                pltpu.VMEM((L,), jnp.int32),
                pltpu.VMEM((L,), jnp.int32),
                pltpu.SemaphoreType.DMA,
                pltpu.SemaphoreType.DMA,
                pltpu.SemaphoreType.DMA,
                pltpu.SemaphoreType.DMA,
            ]
        ),
    )
    def k(xe_h, el_h, xd_h, xt_h, tl_h, prev_h, ye_h, yd_h, p_h, *scr):
        bufs = scr[:ROWS]
        yes = scr[ROWS:ROWS + OSLOTS]
        ps = scr[ROWS + OSLOTS:ROWS + 2 * OSLOTS]
        yds = scr[ROWS + 2 * OSLOTS:2 * ROWS + 2 * OSLOTS]
        el_v, tl_v, sem_lens, sem_in, sem_out, sem_yd = scr[-6:]

        wid = lax.axis_index("s") * NC + lax.axis_index("c")
        base = wid * ROWS
        lanes = lax.iota(jnp.int32, L)

        lens_handles = (
            pltpu.async_copy(el_h.at[pl.ds(base, ROWS)],
                             el_v.at[pl.ds(0, ROWS)], sem_lens),
            pltpu.async_copy(tl_h.at[pl.ds(base, ROWS)],
                             tl_v.at[pl.ds(0, ROWS)], sem_lens),
        )
        in_flight = []
        for r in range(ROWS):
            buf = bufs[r]
            row = base + r
            in_flight.append((
                pltpu.async_copy(prev_h.at[row], buf.at[pl.ds(0, WENC)],
                                 sem_in),
                pltpu.async_copy(xe_h.at[row], buf.at[pl.ds(WENC, MAXS)],
                                 sem_in),
                pltpu.async_copy(xt_h.at[row],
                                 buf.at[pl.ds(WENC + MAXS, MAXS)], sem_in),
            ))
        xd_handles = [
            pltpu.async_copy(xd_h.at[base + r], yds[r].at[pl.ds(0, MAXS)],
                             sem_yd)
            for r in range(ROWS)
        ]

        zv = jnp.zeros((L,), jnp.int32)
        for r in range(ROWS):
            ydr = yds[r]

            @pl.loop(0, (WDEC - MAXS) // L)
            def _(i):
                ydr[pl.ds(pl.multiple_of(MAXS + i * L, L), L)] = zv

        for h in lens_handles:
            h.wait()
        el_vec = el_v[...]
        tl_vec = tl_v[...]

        out_flight = {}
        yd_out = []
        for r in range(ROWS):
            slot = r % OSLOTS
            row = base + r
            if r - OSLOTS in out_flight:
                for h in out_flight.pop(r - OSLOTS):
                    h.wait()
            for h in in_flight[r]:
                h.wait()

            el = el_vec[r]
            tl = tl_vec[r]
            el_lanes = el + lanes
            tl_lanes = tl + lanes
            a_vec = tl_lanes + el
            b_vec = tl_lanes + MAXS
            buf = bufs[r]
            yev = yes[slot]
            pv = ps[slot]

            @pl.loop(0, CHUNKS, unroll=8)
            def _(kk):
                off = pl.multiple_of(kk * L, L)
                yev[pl.ds(off, L)] = plsc.load_gather(buf, [el_lanes + off])
                q = tl_lanes + off
                idx2 = jnp.where(q < WENC, a_vec + off, b_vec + off)
                pv[pl.ds(off, L)] = plsc.load_gather(buf, [idx2])

            xd_handles[r].wait()
            yd_out.append(pltpu.async_copy(yds[r], yd_h.at[row], sem_yd))
            out_flight[r] = (
                pltpu.async_copy(yev, ye_h.at[row], sem_out),
                pltpu.async_copy(pv, p_h.at[row], sem_out),
            )

        for r, hs in sorted(out_flight.items()):
            for h in hs:
                h.wait()
        for h in yd_out:
            h.wait()

    return k(xe, xe_lens, xd, xt, xt_lens, prev)


def kernel(xe, xe_lens, xd, xd_lens, xt, xt_lens, prev):
    el = xe_lens.astype(jnp.int32)
    dl = xd_lens.astype(jnp.int32)
    ye, yd, p = _frames_sc(xe, el, xd, xt, xt_lens.astype(jnp.int32), prev)
    return (ye, el, yd, dl, p)


# speedup vs baseline: 1.9597x; 1.0169x over previous
import functools

import jax
import jax.numpy as jnp
from jax import lax
from jax.experimental import pallas as pl
from jax.experimental.pallas import tpu as pltpu
from jax.experimental.pallas import tpu_sc as plsc

B = 256
WENC = 4096
WDEC = 4096
MAXS = 2048
NC = 2
NS = 16
NW = NC * NS
ROWS = B // NW
BUF = WENC + 2 * MAXS
L = 16
CHUNKS = WENC // L
OSLOTS = 3


def _frames_sc(xe, xe_lens, xd, xt, xt_lens, prev):
    mesh = plsc.VectorSubcoreMesh(core_axis_name="c", subcore_axis_name="s")

    @functools.partial(
        pl.kernel,
        mesh=mesh,
        compiler_params=pltpu.CompilerParams(needs_layout_passes=False),
        out_type=[
            jax.ShapeDtypeStruct((B, WENC), jnp.int32),
            jax.ShapeDtypeStruct((B, WDEC), jnp.int32),
            jax.ShapeDtypeStruct((B, WENC), jnp.int32),
        ],
        scratch_types=(
            [pltpu.VMEM((BUF,), jnp.int32) for _ in range(ROWS)]
            + [pltpu.VMEM((WENC,), jnp.int32) for _ in range(OSLOTS)]
            + [pltpu.VMEM((WENC,), jnp.int32) for _ in range(OSLOTS)]
            + [pltpu.VMEM((WDEC,), jnp.int32) for _ in range(ROWS)]
            + [
                pltpu.VMEM((L,), jnp.int32),
                pltpu.VMEM((L,), jnp.int32),
                pltpu.SemaphoreType.DMA,
                pltpu.SemaphoreType.DMA,
                pltpu.SemaphoreType.DMA,
                pltpu.SemaphoreType.DMA,
            ]
        ),
    )
    def k(xe_h, el_h, xd_h, xt_h, tl_h, prev_h, ye_h, yd_h, p_h, *scr):
        bufs = scr[:ROWS]
        yes = scr[ROWS:ROWS + OSLOTS]
        ps = scr[ROWS + OSLOTS:ROWS + 2 * OSLOTS]
        yds = scr[ROWS + 2 * OSLOTS:2 * ROWS + 2 * OSLOTS]
        el_v, tl_v, sem_lens, sem_in, sem_out, sem_yd = scr[-6:]

        wid = lax.axis_index("s") * NC + lax.axis_index("c")
        base = wid * ROWS
        lanes = lax.iota(jnp.int32, L)

        lens_handles = (
            pltpu.async_copy(el_h.at[pl.ds(base, ROWS)],
                             el_v.at[pl.ds(0, ROWS)], sem_lens),
            pltpu.async_copy(tl_h.at[pl.ds(base, ROWS)],
                             tl_v.at[pl.ds(0, ROWS)], sem_lens),
        )
        in_flight = []
        for r in range(ROWS):
            buf = bufs[r]
            row = base + r
            in_flight.append((
                pltpu.async_copy(prev_h.at[row], buf.at[pl.ds(0, WENC)],
                                 sem_in),
                pltpu.async_copy(xe_h.at[row], buf.at[pl.ds(WENC, MAXS)],
                                 sem_in),
                pltpu.async_copy(xt_h.at[row],
                                 buf.at[pl.ds(WENC + MAXS, MAXS)], sem_in),
            ))
        xd_handles = [
            pltpu.async_copy(xd_h.at[base + r], yds[r].at[pl.ds(0, MAXS)],
                             sem_yd)
            for r in range(ROWS)
        ]

        zv = jnp.zeros((L,), jnp.int32)
        for r in range(ROWS):
            ydr = yds[r]

            @pl.loop(0, (WDEC - MAXS) // L)
            def _(i):
                ydr[pl.ds(pl.multiple_of(MAXS + i * L, L), L)] = zv

        for h in lens_handles:
            h.wait()
        el_vec = el_v[...]
        tl_vec = tl_v[...]

        out_flight = {}
        yd_out = []
        for r in range(ROWS):
            slot = r % OSLOTS
            row = base + r
            if r - OSLOTS in out_flight:
                for h in out_flight.pop(r - OSLOTS):
                    h.wait()
            for h in in_flight[r]:
                h.wait()

            el = el_vec[r]
            tl = tl_vec[r]
            el_lanes = el + lanes
            tl_lanes = tl + lanes
            a_vec = tl_lanes + el
            b_vec = tl_lanes + MAXS
            buf = bufs[r]
            yev = yes[slot]
            pv = ps[slot]

            @pl.loop(0, CHUNKS, unroll=4)
            def _(kk):
                off = pl.multiple_of(kk * L, L)
                yev[pl.ds(off, L)] = plsc.load_gather(buf, [el_lanes + off])
                q = tl_lanes + off
                idx2 = jnp.where(q < WENC, a_vec + off, b_vec + off)
                pv[pl.ds(off, L)] = plsc.load_gather(buf, [idx2])

            xd_handles[r].wait()
            yd_out.append(pltpu.async_copy(yds[r], yd_h.at[row], sem_yd))
            out_flight[r] = (
                pltpu.async_copy(yev, ye_h.at[row], sem_out),
                pltpu.async_copy(pv, p_h.at[row], sem_out),
            )

        for r, hs in sorted(out_flight.items()):
            for h in hs:
                h.wait()
        for h in yd_out:
            h.wait()

    return k(xe, xe_lens, xd, xt, xt_lens, prev)


def kernel(xe, xe_lens, xd, xd_lens, xt, xt_lens, prev):
    el = xe_lens.astype(jnp.int32)
    dl = xd_lens.astype(jnp.int32)
    ye, yd, p = _frames_sc(xe, el, xd, xt, xt_lens.astype(jnp.int32), prev)
    return (ye, el, yd, dl, p)


# speedup vs baseline: 1.9709x; 1.0057x over previous
import functools

import jax
import jax.numpy as jnp
from jax import lax
from jax.experimental import pallas as pl
from jax.experimental.pallas import tpu as pltpu
from jax.experimental.pallas import tpu_sc as plsc

B = 256
WENC = 4096
WDEC = 4096
MAXS = 2048
NC = 2
NS = 16
NW = NC * NS
ROWS = B // NW
BUF = WENC + 2 * MAXS
L = 16
CHUNKS = WENC // L
OSLOTS = 3


def _frames_sc(xe, xe_lens, xd, xt, xt_lens, prev):
    mesh = plsc.VectorSubcoreMesh(core_axis_name="c", subcore_axis_name="s")

    @functools.partial(
        pl.kernel,
        mesh=mesh,
        compiler_params=pltpu.CompilerParams(needs_layout_passes=False),
        out_type=[
            jax.ShapeDtypeStruct((B, WENC), jnp.int32),
            jax.ShapeDtypeStruct((B, WDEC), jnp.int32),
            jax.ShapeDtypeStruct((B, WENC), jnp.int32),
        ],
        scratch_types=(
            [pltpu.VMEM((BUF,), jnp.int32) for _ in range(ROWS)]
            + [pltpu.VMEM((WENC,), jnp.int32) for _ in range(OSLOTS)]
            + [pltpu.VMEM((WENC,), jnp.int32) for _ in range(OSLOTS)]
            + [pltpu.VMEM((WDEC,), jnp.int32) for _ in range(ROWS)]
            + [
                pltpu.VMEM((L,), jnp.int32),
                pltpu.VMEM((L,), jnp.int32),
                pltpu.SemaphoreType.DMA,
                pltpu.SemaphoreType.DMA,
                pltpu.SemaphoreType.DMA,
                pltpu.SemaphoreType.DMA,
            ]
        ),
    )
    def k(xe_h, el_h, xd_h, xt_h, tl_h, prev_h, ye_h, yd_h, p_h, *scr):
        bufs = scr[:ROWS]
        yes = scr[ROWS:ROWS + OSLOTS]
        ps = scr[ROWS + OSLOTS:ROWS + 2 * OSLOTS]
        yds = scr[ROWS + 2 * OSLOTS:2 * ROWS + 2 * OSLOTS]
        el_v, tl_v, sem_lens, sem_in, sem_out, sem_yd = scr[-6:]

        wid = lax.axis_index("s") * NC + lax.axis_index("c")
        base = wid * ROWS
        lanes = lax.iota(jnp.int32, L)

        lens_handles = (
            pltpu.async_copy(el_h.at[pl.ds(base, ROWS)],
                             el_v.at[pl.ds(0, ROWS)], sem_lens),
            pltpu.async_copy(tl_h.at[pl.ds(base, ROWS)],
                             tl_v.at[pl.ds(0, ROWS)], sem_lens),
        )
        in_flight = []
        for r in range(ROWS):
            buf = bufs[r]
            row = base + r
            in_flight.append((
                pltpu.async_copy(prev_h.at[row], buf.at[pl.ds(0, WENC)],
                                 sem_in),
                pltpu.async_copy(xe_h.at[row], buf.at[pl.ds(WENC, MAXS)],
                                 sem_in),
                pltpu.async_copy(xt_h.at[row],
                                 buf.at[pl.ds(WENC + MAXS, MAXS)], sem_in),
            ))
        xd_handles = [
            pltpu.async_copy(xd_h.at[base + r], yds[r].at[pl.ds(0, MAXS)],
                             sem_yd)
            for r in range(ROWS)
        ]

        zv = jnp.zeros((L,), jnp.int32)
        for r in range(ROWS):
            ydr = yds[r]

            @pl.loop(0, (WDEC - MAXS) // L)
            def _(i):
                ydr[pl.ds(pl.multiple_of(MAXS + i * L, L), L)] = zv

        for h in lens_handles:
            h.wait()
        el_vec = el_v[...]
        tl_vec = tl_v[...]

        out_flight = {}
        yd_out = []
        for r in range(ROWS):
            slot = r % OSLOTS
            row = base + r
            if r - OSLOTS in out_flight:
                for h in out_flight.pop(r - OSLOTS):
                    h.wait()
            for h in in_flight[r]:
                h.wait()

            el = el_vec[r]
            tl = tl_vec[r]
            el_lanes = el + lanes
            tl_lanes = tl + lanes
            a_vec = tl_lanes + el
            b_vec = tl_lanes + MAXS
            buf = bufs[r]
            yev = yes[slot]
            pv = ps[slot]

            @pl.loop(0, CHUNKS, unroll=2)
            def _(kk):
                off = pl.multiple_of(kk * L, L)
                yev[pl.ds(off, L)] = plsc.load_gather(buf, [el_lanes + off])
                q = tl_lanes + off
                idx2 = jnp.where(q < WENC, a_vec + off, b_vec + off)
                pv[pl.ds(off, L)] = plsc.load_gather(buf, [idx2])

            xd_handles[r].wait()
            yd_out.append(pltpu.async_copy(yds[r], yd_h.at[row], sem_yd))
            out_flight[r] = (
                pltpu.async_copy(yev, ye_h.at[row], sem_out),
                pltpu.async_copy(pv, p_h.at[row], sem_out),
            )

        for r, hs in sorted(out_flight.items()):
            for h in hs:
                h.wait()
        for h in yd_out:
            h.wait()

    return k(xe, xe_lens, xd, xt, xt_lens, prev)


def kernel(xe, xe_lens, xd, xd_lens, xt, xt_lens, prev):
    el = xe_lens.astype(jnp.int32)
    dl = xd_lens.astype(jnp.int32)
    ye, yd, p = _frames_sc(xe, el, xd, xt, xt_lens.astype(jnp.int32), prev)
    return (ye, el, yd, dl, p)


# speedup vs baseline: 2.1382x; 1.0849x over previous
import functools

import jax
import jax.numpy as jnp
from jax import lax
from jax.experimental import pallas as pl
from jax.experimental.pallas import tpu as pltpu
from jax.experimental.pallas import tpu_sc as plsc

B = 256
WENC = 4096
WDEC = 4096
MAXS = 2048
NC = 2
NS = 16
NW = NC * NS
ROWS = B // NW
BUF = WENC + 2 * MAXS
L = 16
CHUNKS = WENC // L
OSLOTS = 4
YD_BR = 32


def _frames_sc(xe, xe_lens, xt, xt_lens, prev):
    mesh = plsc.VectorSubcoreMesh(core_axis_name="c", subcore_axis_name="s")

    @functools.partial(
        pl.kernel,
        mesh=mesh,
        compiler_params=pltpu.CompilerParams(needs_layout_passes=False),
        out_type=[
            jax.ShapeDtypeStruct((B, WENC), jnp.int32),
            jax.ShapeDtypeStruct((B, WENC), jnp.int32),
        ],
        scratch_types=(
            [pltpu.VMEM((BUF,), jnp.int32) for _ in range(ROWS)]
            + [pltpu.VMEM((WENC,), jnp.int32) for _ in range(OSLOTS)]
            + [pltpu.VMEM((WENC,), jnp.int32) for _ in range(OSLOTS)]
            + [
                pltpu.VMEM((L,), jnp.int32),
                pltpu.VMEM((L,), jnp.int32),
                pltpu.SemaphoreType.DMA,
                pltpu.SemaphoreType.DMA,
                pltpu.SemaphoreType.DMA,
            ]
        ),
    )
    def k(xe_h, el_h, xt_h, tl_h, prev_h, ye_h, p_h, *scr):
        bufs = scr[:ROWS]
        yes = scr[ROWS:ROWS + OSLOTS]
        ps = scr[ROWS + OSLOTS:ROWS + 2 * OSLOTS]
        el_v, tl_v, sem_lens, sem_in, sem_out = scr[-5:]

        wid = lax.axis_index("s") * NC + lax.axis_index("c")
        base = wid * ROWS
        lanes = lax.iota(jnp.int32, L)

        lens_handles = (
            pltpu.async_copy(el_h.at[pl.ds(base, ROWS)],
                             el_v.at[pl.ds(0, ROWS)], sem_lens),
            pltpu.async_copy(tl_h.at[pl.ds(base, ROWS)],
                             tl_v.at[pl.ds(0, ROWS)], sem_lens),
        )
        in_flight = []
        for r in range(ROWS):
            buf = bufs[r]
            row = base + r
            in_flight.append((
                pltpu.async_copy(prev_h.at[row], buf.at[pl.ds(0, WENC)],
                                 sem_in),
                pltpu.async_copy(xe_h.at[row], buf.at[pl.ds(WENC, MAXS)],
                                 sem_in),
                pltpu.async_copy(xt_h.at[row],
                                 buf.at[pl.ds(WENC + MAXS, MAXS)], sem_in),
            ))

        for h in lens_handles:
            h.wait()
        el_vec = el_v[...]
        tl_vec = tl_v[...]

        out_flight = {}
        for r in range(ROWS):
            slot = r % OSLOTS
            row = base + r
            if r - OSLOTS in out_flight:
                for h in out_flight.pop(r - OSLOTS):
                    h.wait()
            for h in in_flight[r]:
                h.wait()

            el = el_vec[r]
            tl = tl_vec[r]
            el_lanes = el + lanes
            tl_lanes = tl + lanes
            a_vec = tl_lanes + el
            b_vec = tl_lanes + MAXS
            buf = bufs[r]
            yev = yes[slot]
            pv = ps[slot]

            @pl.loop(0, CHUNKS, unroll=2)
            def _(kk):
                off = pl.multiple_of(kk * L, L)
                yev[pl.ds(off, L)] = plsc.load_gather(buf, [el_lanes + off])
                q = tl_lanes + off
                idx2 = jnp.where(q < WENC, a_vec + off, b_vec + off)
                pv[pl.ds(off, L)] = plsc.load_gather(buf, [idx2])

            out_flight[r] = (
                pltpu.async_copy(yev, ye_h.at[row], sem_out),
                pltpu.async_copy(pv, p_h.at[row], sem_out),
            )

        for r, hs in sorted(out_flight.items()):
            for h in hs:
                h.wait()

    return k(xe, xe_lens, xt, xt_lens, prev)


def _pad_tc(xd):
    def body(x_ref, o_ref):
        o_ref[:, :MAXS] = x_ref[...]
        o_ref[:, MAXS:] = jnp.zeros((YD_BR, WDEC - MAXS), jnp.int32)

    return pl.pallas_call(
        body,
        grid=(B // YD_BR,),
        in_specs=[pl.BlockSpec((YD_BR, MAXS), lambda i: (i, 0))],
        out_specs=pl.BlockSpec((YD_BR, WDEC), lambda i: (i, 0)),
        out_shape=jax.ShapeDtypeStruct((B, WDEC), jnp.int32),
    )(xd)


def kernel(xe, xe_lens, xd, xd_lens, xt, xt_lens, prev):
    el = xe_lens.astype(jnp.int32)
    dl = xd_lens.astype(jnp.int32)
    ye, p = _frames_sc(xe, el, xt, xt_lens.astype(jnp.int32), prev)
    yd = _pad_tc(xd)
    return (ye, el, yd, dl, p)
